# baseline (speedup 1.0000x reference)
import functools

import jax
import jax.numpy as jnp
import numpy as np
from jax.experimental import pallas as pl
from jax.experimental.pallas import tpu as pltpu
from jax.experimental.pallas import tpu_sc as plsc

N_NODES = 10000
N_EDGES = 320000
D_FEAT = 128
D_EDGE = 16
D_HID = 128
D_OUT = 16

_W = 128
_NPAD = 327680
_GRID = _NPAD // _W
_B3 = 2000
_LANES = 16
_HALF = D_HID // 2

_PERM = np.empty(D_HID, dtype=np.int32)
_PERM[0::2] = np.arange(_HALF)
_PERM[1::2] = np.arange(_HALF) + _HALF

_NWORK = 32
_WIN_PER = _GRID // _NWORK


def _node_proj(node_attr, w1ab):
    def body(n_ref, w_ref, p_ref, q_ref):
        n = n_ref[...]
        p_ref[...] = jnp.dot(n, w_ref[:D_FEAT, :],
                             preferred_element_type=jnp.float32)
        q_ref[...] = jnp.dot(n, w_ref[D_FEAT:, :],
                             preferred_element_type=jnp.float32)

    return pl.pallas_call(
        body,
        out_shape=[
            jax.ShapeDtypeStruct((N_NODES, D_HID), jnp.float32),
            jax.ShapeDtypeStruct((N_NODES, D_HID), jnp.float32),
        ],
    )(node_attr, w1ab)


def _sc_gather_add(p, q, sidx, ridx):
    mesh = plsc.VectorSubcoreMesh(core_axis_name="c", subcore_axis_name="s")
    cp = pltpu.CompilerParams(needs_layout_passes=False)

    @functools.partial(
        pl.kernel,
        out_type=jax.ShapeDtypeStruct((_NPAD // 2, D_HID), jnp.int32),
        mesh=mesh,
        compiler_params=cp,
        scratch_types=[
            pltpu.VMEM((2, _W), jnp.int32),
            pltpu.VMEM((2, _W), jnp.int32),
            pltpu.VMEM((2, _W, D_HID), jnp.float32),
            pltpu.VMEM((2, _W, D_HID), jnp.float32),
            pltpu.VMEM((2, _W // 2, D_HID), jnp.int32),
            pltpu.SemaphoreType.DMA,
            pltpu.SemaphoreType.DMA,
            pltpu.SemaphoreType.DMA,
            pltpu.SemaphoreType.DMA,
            pltpu.SemaphoreType.DMA,
            pltpu.SemaphoreType.DMA,
        ],
    )
    def k(p_hbm, q_hbm, si_hbm, ri_hbm, o_hbm, si_v, ri_v, gp_v, gq_v, out_v,
          s_i0, s_i1, s_g0, s_g1, s_o0, s_o1):
        sem_i = (s_i0, s_i1)
        sem_g = (s_g0, s_g1)
        sem_o = (s_o0, s_o1)
        wid = jax.lax.axis_index("s") * 2 + jax.lax.axis_index("c")
        w0 = wid * _WIN_PER

        def fire_idx(j, b):
            pltpu.make_async_copy(si_hbm.at[w0 + j], si_v.at[b], sem_i[b]).start()
            pltpu.make_async_copy(ri_hbm.at[w0 + j], ri_v.at[b], sem_i[b]).start()

        def wait_idx(j, b):
            pltpu.make_async_copy(si_hbm.at[w0 + j], si_v.at[b], sem_i[b]).wait()
            pltpu.make_async_copy(ri_hbm.at[w0 + j], ri_v.at[b], sem_i[b]).wait()

        def fire_gathers(b):
            pltpu.make_async_copy(p_hbm.at[si_v.at[b]], gp_v.at[b], sem_g[b]).start()
            pltpu.make_async_copy(q_hbm.at[ri_v.at[b]], gq_v.at[b], sem_g[b]).start()

        def wait_gathers(b):
            pltpu.make_async_copy(p_hbm.at[si_v.at[b]], gp_v.at[b], sem_g[b]).wait()
            pltpu.make_async_copy(q_hbm.at[ri_v.at[b]], gq_v.at[b], sem_g[b]).wait()

        def out_copy(j, b):
            return pltpu.make_async_copy(
                out_v.at[b],
                o_hbm.at[pl.ds((w0 + j) * (_W // 2), _W // 2), :],
                sem_o[b])

        def rne_bf16(u):
            lsb = (u >> 16) & jnp.uint32(1)
            return (u + lsb + jnp.uint32(0x7FFF)) >> 16

        fire_idx(0, 0)
        wait_idx(0, 0)
        fire_gathers(0)
        fire_idx(1, 1)

        @pl.loop(0, _WIN_PER, step=2)
        def _(j0):
            for b in (0, 1):
                j = j0 + b
                nb = 1 - b

                @pl.when(jnp.logical_and(j >= 1, j + 1 < _WIN_PER))
                def _():
                    out_copy(j - 1, nb).wait()

                @pl.when(j + 1 < _WIN_PER)
                def _():
                    wait_idx(j + 1, nb)
                    fire_gathers(nb)

                wait_gathers(b)

                @pl.when(j + 2 < _WIN_PER)
                def _():
                    fire_idx(j + 2, b)

                @pl.loop(0, _W // 2)
                def _(r):
                    for half in (0, 1):
                        e = 2 * r + half
                        for cg in range(0, _HALF, _LANES):
                            lo = pl.ds(cg, _LANES)
                            hi = pl.ds(_HALF + cg, _LANES)
                            s_lo = gp_v.at[b, e, lo][...] + gq_v.at[b, e, lo][...]
                            s_hi = gp_v.at[b, e, hi][...] + gq_v.at[b, e, hi][...]
                            u_lo = rne_bf16(plsc.bitcast(s_lo, jnp.uint32))
                            u_hi = rne_bf16(plsc.bitcast(s_hi, jnp.uint32))
                            word = u_lo | (u_hi << 16)
                            out_v.at[b, r, pl.ds(half * _HALF + cg, _LANES)][...] = (
                                plsc.bitcast(word, jnp.int32))

                out_copy(j, b).start()

        out_copy(_WIN_PER - 2, 0).wait()
        out_copy(_WIN_PER - 1, 1).wait()

    return k(p, q, sidx, ridx)


def _epilogue(g, edge_attr, w1c, b1, w2, b2):
    def body(g_ref, e_ref, w1c_ref, b1_ref, w2_ref, b2_ref, o_ref):
        pre = (g_ref[...].astype(jnp.float32)
               + jnp.dot(e_ref[...], w1c_ref[...],
                         preferred_element_type=jnp.float32)
               + b1_ref[...])
        h = jnp.maximum(pre, 0.0)
        o_ref[...] = (jnp.dot(h, w2_ref[...],
                              preferred_element_type=jnp.float32)
                      + b2_ref[...])

    return pl.pallas_call(
        body,
        grid=(N_EDGES // _B3,),
        in_specs=[
            pl.BlockSpec((_B3, D_HID), lambda i: (i, 0)),
            pl.BlockSpec((_B3, D_EDGE), lambda i: (i, 0)),
            pl.BlockSpec((D_EDGE, D_HID), lambda i: (0, 0)),
            pl.BlockSpec((1, D_HID), lambda i: (0, 0)),
            pl.BlockSpec((D_HID, D_OUT), lambda i: (0, 0)),
            pl.BlockSpec((1, D_OUT), lambda i: (0, 0)),
        ],
        out_specs=pl.BlockSpec((_B3, D_OUT), lambda i: (i, 0)),
        out_shape=jax.ShapeDtypeStruct((N_EDGES, D_OUT), jnp.float32),
    )(g, edge_attr, w1c, b1, w2, b2)


def kernel(node_attr, edge_attr, edge_index, edge_contact_attr,
           edge_contact_index, W1, b1, W2, b2):
    idx = edge_index.astype(jnp.int32)
    idx = jnp.pad(idx, ((0, 0), (0, _NPAD - N_EDGES)))
    sidx = idx[0].reshape(_GRID, _W)
    ridx = idx[1].reshape(_GRID, _W)

    p, q = _node_proj(node_attr, W1[: 2 * D_FEAT, :])
    g_i32 = _sc_gather_add(p, q, sidx, ridx)
    g = jax.lax.bitcast_convert_type(g_i32, jnp.bfloat16).reshape(_NPAD, D_HID)

    perm = jnp.asarray(_PERM)
    w1c_p = W1[2 * D_FEAT:, :][:, perm]
    b1_p = b1[perm].reshape(1, D_HID)
    w2_p = W2[perm, :]
    out = _epilogue(g, edge_attr, w1c_p, b1_p, w2_p, b2.reshape(1, D_OUT))
    return (node_attr, out, edge_index, edge_contact_attr, edge_contact_index)


# speedup vs baseline: 18.9286x; 18.9286x over previous
import functools

import jax
import jax.numpy as jnp
import numpy as np
from jax.experimental import pallas as pl
from jax.experimental.pallas import tpu as pltpu
from jax.experimental.pallas import tpu_sc as plsc

N_NODES = 10000
N_EDGES = 320000
D_FEAT = 128
D_EDGE = 16
D_HID = 128
D_OUT = 16

_W = 128
_NPAD = 327680
_GRID = _NPAD // _W
_B3 = 2000
_LANES = 16
_HALF = D_HID // 2

_EF = np.concatenate([np.arange(0, D_HID, 2), np.arange(1, D_HID, 2)])

_NWORK = 32
_WIN_PER = _GRID // _NWORK


def _node_proj(node_attr, w1ab):
    def body(n_ref, w_ref, p_ref, q_ref):
        n = n_ref[...]
        p_ref[...] = jnp.dot(n, w_ref[:D_FEAT, :],
                             preferred_element_type=jnp.float32)
        q_ref[...] = jnp.dot(n, w_ref[D_FEAT:, :],
                             preferred_element_type=jnp.float32)

    return pl.pallas_call(
        body,
        out_shape=[
            jax.ShapeDtypeStruct((N_NODES, D_HID), jnp.float32),
            jax.ShapeDtypeStruct((N_NODES, D_HID), jnp.float32),
        ],
    )(node_attr, w1ab)


def _sc_gather_add(p, q, sidx, ridx):
    mesh = plsc.VectorSubcoreMesh(core_axis_name="c", subcore_axis_name="s")
    cp = pltpu.CompilerParams(needs_layout_passes=False)

    @functools.partial(
        pl.kernel,
        out_type=jax.ShapeDtypeStruct((_NPAD, _HALF), jnp.int32),
        mesh=mesh,
        compiler_params=cp,
        scratch_types=[
            pltpu.VMEM((2, _W), jnp.int32),
            pltpu.VMEM((2, _W), jnp.int32),
            pltpu.VMEM((2, _W, D_HID), jnp.float32),
            pltpu.VMEM((2, _W, D_HID), jnp.float32),
            pltpu.VMEM((2, _W, _HALF), jnp.int32),
            pltpu.SemaphoreType.DMA,
            pltpu.SemaphoreType.DMA,
            pltpu.SemaphoreType.DMA,
            pltpu.SemaphoreType.DMA,
            pltpu.SemaphoreType.DMA,
            pltpu.SemaphoreType.DMA,
        ],
    )
    def k(p_hbm, q_hbm, si_hbm, ri_hbm, o_hbm, si_v, ri_v, gp_v, gq_v, out_v,
          s_i0, s_i1, s_g0, s_g1, s_o0, s_o1):
        sem_i = (s_i0, s_i1)
        sem_g = (s_g0, s_g1)
        sem_o = (s_o0, s_o1)
        wid = jax.lax.axis_index("s") * 2 + jax.lax.axis_index("c")
        w0 = wid * _WIN_PER

        def fire_idx(j, b):
            pltpu.make_async_copy(si_hbm.at[w0 + j], si_v.at[b], sem_i[b]).start()
            pltpu.make_async_copy(ri_hbm.at[w0 + j], ri_v.at[b], sem_i[b]).start()

        def wait_idx(j, b):
            pltpu.make_async_copy(si_hbm.at[w0 + j], si_v.at[b], sem_i[b]).wait()
            pltpu.make_async_copy(ri_hbm.at[w0 + j], ri_v.at[b], sem_i[b]).wait()

        def fire_gathers(b):
            pltpu.make_async_copy(p_hbm.at[si_v.at[b]], gp_v.at[b], sem_g[b]).start()
            pltpu.make_async_copy(q_hbm.at[ri_v.at[b]], gq_v.at[b], sem_g[b]).start()

        def wait_gathers(b):
            pltpu.make_async_copy(p_hbm.at[si_v.at[b]], gp_v.at[b], sem_g[b]).wait()
            pltpu.make_async_copy(q_hbm.at[ri_v.at[b]], gq_v.at[b], sem_g[b]).wait()

        def out_copy(j, b):
            return pltpu.make_async_copy(
                out_v.at[b],
                o_hbm.at[pl.ds((w0 + j) * _W, _W), :],
                sem_o[b])

        def rne_bf16(u):
            lsb = (u >> 16) & jnp.uint32(1)
            return (u + lsb + jnp.uint32(0x7FFF)) >> 16

        fire_idx(0, 0)
        wait_idx(0, 0)
        fire_gathers(0)
        fire_idx(1, 1)

        @pl.loop(0, _WIN_PER, step=2)
        def _(j0):
            for b in (0, 1):
                j = j0 + b
                nb = 1 - b

                @pl.when(jnp.logical_and(j >= 1, j + 1 < _WIN_PER))
                def _():
                    out_copy(j - 1, nb).wait()

                @pl.when(j + 1 < _WIN_PER)
                def _():
                    wait_idx(j + 1, nb)
                    fire_gathers(nb)

                wait_gathers(b)

                @pl.when(j + 2 < _WIN_PER)
                def _():
                    fire_idx(j + 2, b)

                @pl.loop(0, _W)
                def _(e):
                    for cg in range(0, _HALF, _LANES):
                        lo = pl.ds(cg, _LANES)
                        hi = pl.ds(_HALF + cg, _LANES)
                        s_lo = gp_v.at[b, e, lo][...] + gq_v.at[b, e, lo][...]
                        s_hi = gp_v.at[b, e, hi][...] + gq_v.at[b, e, hi][...]
                        u_lo = rne_bf16(plsc.bitcast(s_lo, jnp.uint32))
                        u_hi = rne_bf16(plsc.bitcast(s_hi, jnp.uint32))
                        word = u_lo | (u_hi << 16)
                        out_v.at[b, e, lo][...] = plsc.bitcast(word, jnp.int32)

                out_copy(j, b).start()

        out_copy(_WIN_PER - 2, 0).wait()
        out_copy(_WIN_PER - 1, 1).wait()

    return k(p, q, sidx, ridx)


def _epilogue(g32, edge_attr, w1c_e, w1c_o, b1_e, b1_o, w2_e, w2_o, b2):
    def body(g_ref, e_ref, w1ce_ref, w1co_ref, b1e_ref, b1o_ref,
             w2e_ref, w2o_ref, b2_ref, o_ref):
        w = g_ref[...]
        x_even = jax.lax.bitcast_convert_type(
            jnp.left_shift(w, 16), jnp.float32)
        x_odd = jax.lax.bitcast_convert_type(
            jnp.bitwise_and(w, jnp.int32(-65536)), jnp.float32)
        e = e_ref[...]
        pre_e = x_even + jnp.dot(e, w1ce_ref[...],
                                 preferred_element_type=jnp.float32) + b1e_ref[...]
        pre_o = x_odd + jnp.dot(e, w1co_ref[...],
                                preferred_element_type=jnp.float32) + b1o_ref[...]
        h_e = jnp.maximum(pre_e, 0.0)
        h_o = jnp.maximum(pre_o, 0.0)
        o_ref[...] = (jnp.dot(h_e, w2e_ref[...],
                              preferred_element_type=jnp.float32)
                      + jnp.dot(h_o, w2o_ref[...],
                                preferred_element_type=jnp.float32)
                      + b2_ref[...])

    return pl.pallas_call(
        body,
        grid=(N_EDGES // _B3,),
        in_specs=[
            pl.BlockSpec((_B3, _HALF), lambda i: (i, 0)),
            pl.BlockSpec((_B3, D_EDGE), lambda i: (i, 0)),
            pl.BlockSpec((D_EDGE, _HALF), lambda i: (0, 0)),
            pl.BlockSpec((D_EDGE, _HALF), lambda i: (0, 0)),
            pl.BlockSpec((1, _HALF), lambda i: (0, 0)),
            pl.BlockSpec((1, _HALF), lambda i: (0, 0)),
            pl.BlockSpec((_HALF, D_OUT), lambda i: (0, 0)),
            pl.BlockSpec((_HALF, D_OUT), lambda i: (0, 0)),
            pl.BlockSpec((1, D_OUT), lambda i: (0, 0)),
        ],
        out_specs=pl.BlockSpec((_B3, D_OUT), lambda i: (i, 0)),
        out_shape=jax.ShapeDtypeStruct((N_EDGES, D_OUT), jnp.float32),
    )(g32, edge_attr, w1c_e, w1c_o, b1_e, b1_o, w2_e, w2_o, b2)


def kernel(node_attr, edge_attr, edge_index, edge_contact_attr,
           edge_contact_index, W1, b1, W2, b2):
    idx = edge_index.astype(jnp.int32)
    idx = jnp.pad(idx, ((0, 0), (0, _NPAD - N_EDGES)))
    sidx = idx[0].reshape(_GRID, _W)
    ridx = idx[1].reshape(_GRID, _W)

    ef = jnp.asarray(_EF)
    w1ab_p = W1[: 2 * D_FEAT, :][:, ef]
    p, q = _node_proj(node_attr, w1ab_p)
    g32 = _sc_gather_add(p, q, sidx, ridx)

    w1c = W1[2 * D_FEAT:, :]
    out = _epilogue(
        g32, edge_attr,
        w1c[:, 0::2], w1c[:, 1::2],
        b1[0::2].reshape(1, _HALF), b1[1::2].reshape(1, _HALF),
        W2[0::2, :], W2[1::2, :],
        b2.reshape(1, D_OUT))
    return (node_attr, out, edge_index, edge_contact_attr, edge_contact_index)


# speedup vs baseline: 19.3823x; 1.0240x over previous
import functools

import jax
import jax.numpy as jnp
import numpy as np
from jax.experimental import pallas as pl
from jax.experimental.pallas import tpu as pltpu
from jax.experimental.pallas import tpu_sc as plsc

N_NODES = 10000
N_EDGES = 320000
D_FEAT = 128
D_EDGE = 16
D_HID = 128
D_OUT = 16

_W = 128
_NPAD = 327680
_GRID = _NPAD // _W
_B3 = 8000
_LANES = 16
_HALF = D_HID // 2

_EF = np.concatenate([np.arange(0, D_HID, 2), np.arange(1, D_HID, 2)])

_NWORK = 32
_WIN_PER = _GRID // _NWORK


def _node_proj(node_attr, w1ab):
    def body(n_ref, w_ref, p_ref, q_ref):
        n = n_ref[...]
        p_ref[...] = jnp.dot(n, w_ref[:D_FEAT, :],
                             preferred_element_type=jnp.float32)
        q_ref[...] = jnp.dot(n, w_ref[D_FEAT:, :],
                             preferred_element_type=jnp.float32)

    return pl.pallas_call(
        body,
        out_shape=[
            jax.ShapeDtypeStruct((N_NODES, D_HID), jnp.float32),
            jax.ShapeDtypeStruct((N_NODES, D_HID), jnp.float32),
        ],
    )(node_attr, w1ab)


def _sc_gather_add(p, q, sidx, ridx):
    mesh = plsc.VectorSubcoreMesh(core_axis_name="c", subcore_axis_name="s")
    cp = pltpu.CompilerParams(needs_layout_passes=False)

    @functools.partial(
        pl.kernel,
        out_type=jax.ShapeDtypeStruct((_NPAD, _HALF), jnp.int32),
        mesh=mesh,
        compiler_params=cp,
        scratch_types=[
            pltpu.VMEM((2, _W), jnp.int32),
            pltpu.VMEM((2, _W), jnp.int32),
            pltpu.VMEM((2, _W, D_HID), jnp.float32),
            pltpu.VMEM((2, _W, D_HID), jnp.float32),
            pltpu.VMEM((2, _W, _HALF), jnp.int32),
            pltpu.SemaphoreType.DMA,
            pltpu.SemaphoreType.DMA,
            pltpu.SemaphoreType.DMA,
            pltpu.SemaphoreType.DMA,
            pltpu.SemaphoreType.DMA,
            pltpu.SemaphoreType.DMA,
        ],
    )
    def k(p_hbm, q_hbm, si_hbm, ri_hbm, o_hbm, si_v, ri_v, gp_v, gq_v, out_v,
          s_i0, s_i1, s_g0, s_g1, s_o0, s_o1):
        sem_i = (s_i0, s_i1)
        sem_g = (s_g0, s_g1)
        sem_o = (s_o0, s_o1)
        wid = jax.lax.axis_index("s") * 2 + jax.lax.axis_index("c")
        w0 = wid * _WIN_PER

        def fire_idx(j, b):
            pltpu.make_async_copy(si_hbm.at[w0 + j], si_v.at[b], sem_i[b]).start()
            pltpu.make_async_copy(ri_hbm.at[w0 + j], ri_v.at[b], sem_i[b]).start()

        def wait_idx(j, b):
            pltpu.make_async_copy(si_hbm.at[w0 + j], si_v.at[b], sem_i[b]).wait()
            pltpu.make_async_copy(ri_hbm.at[w0 + j], ri_v.at[b], sem_i[b]).wait()

        def fire_gathers(b):
            pltpu.make_async_copy(p_hbm.at[si_v.at[b]], gp_v.at[b], sem_g[b]).start()
            pltpu.make_async_copy(q_hbm.at[ri_v.at[b]], gq_v.at[b], sem_g[b]).start()

        def wait_gathers(b):
            pltpu.make_async_copy(p_hbm.at[si_v.at[b]], gp_v.at[b], sem_g[b]).wait()
            pltpu.make_async_copy(q_hbm.at[ri_v.at[b]], gq_v.at[b], sem_g[b]).wait()

        def out_copy(j, b):
            return pltpu.make_async_copy(
                out_v.at[b],
                o_hbm.at[pl.ds((w0 + j) * _W, _W), :],
                sem_o[b])

        def rne_bf16(u):
            lsb = (u >> 16) & jnp.uint32(1)
            return (u + lsb + jnp.uint32(0x7FFF)) >> 16

        fire_idx(0, 0)
        wait_idx(0, 0)
        fire_gathers(0)
        fire_idx(1, 1)

        @pl.loop(0, _WIN_PER, step=2)
        def _(j0):
            for b in (0, 1):
                j = j0 + b
                nb = 1 - b

                @pl.when(jnp.logical_and(j >= 1, j + 1 < _WIN_PER))
                def _():
                    out_copy(j - 1, nb).wait()

                @pl.when(j + 1 < _WIN_PER)
                def _():
                    wait_idx(j + 1, nb)
                    fire_gathers(nb)

                wait_gathers(b)

                @pl.when(j + 2 < _WIN_PER)
                def _():
                    fire_idx(j + 2, b)

                @pl.loop(0, _W)
                def _(e):
                    for cg in range(0, _HALF, _LANES):
                        lo = pl.ds(cg, _LANES)
                        hi = pl.ds(_HALF + cg, _LANES)
                        s_lo = gp_v.at[b, e, lo][...] + gq_v.at[b, e, lo][...]
                        s_hi = gp_v.at[b, e, hi][...] + gq_v.at[b, e, hi][...]
                        u_lo = rne_bf16(plsc.bitcast(s_lo, jnp.uint32))
                        u_hi = rne_bf16(plsc.bitcast(s_hi, jnp.uint32))
                        word = u_lo | (u_hi << 16)
                        out_v.at[b, e, lo][...] = plsc.bitcast(word, jnp.int32)

                out_copy(j, b).start()

        out_copy(_WIN_PER - 2, 0).wait()
        out_copy(_WIN_PER - 1, 1).wait()

    return k(p, q, sidx, ridx)


def _epilogue(g32, edge_attr, w1c_e, w1c_o, b1_e, b1_o, w2_e, w2_o, b2):
    def body(g_ref, e_ref, w1ce_ref, w1co_ref, b1e_ref, b1o_ref,
             w2e_ref, w2o_ref, b2_ref, o_ref):
        w = g_ref[...]
        x_even = jax.lax.bitcast_convert_type(
            jnp.left_shift(w, 16), jnp.float32)
        x_odd = jax.lax.bitcast_convert_type(
            jnp.bitwise_and(w, jnp.int32(-65536)), jnp.float32)
        e = e_ref[...]
        pre_e = x_even + jnp.dot(e, w1ce_ref[...],
                                 preferred_element_type=jnp.float32) + b1e_ref[...]
        pre_o = x_odd + jnp.dot(e, w1co_ref[...],
                                preferred_element_type=jnp.float32) + b1o_ref[...]
        h_e = jnp.maximum(pre_e, 0.0)
        h_o = jnp.maximum(pre_o, 0.0)
        o_ref[...] = (jnp.dot(h_e, w2e_ref[...],
                              preferred_element_type=jnp.float32)
                      + jnp.dot(h_o, w2o_ref[...],
                                preferred_element_type=jnp.float32)
                      + b2_ref[...])

    return pl.pallas_call(
        body,
        grid=(N_EDGES // _B3,),
        in_specs=[
            pl.BlockSpec((_B3, _HALF), lambda i: (i, 0)),
            pl.BlockSpec((_B3, D_EDGE), lambda i: (i, 0)),
            pl.BlockSpec((D_EDGE, _HALF), lambda i: (0, 0)),
            pl.BlockSpec((D_EDGE, _HALF), lambda i: (0, 0)),
            pl.BlockSpec((1, _HALF), lambda i: (0, 0)),
            pl.BlockSpec((1, _HALF), lambda i: (0, 0)),
            pl.BlockSpec((_HALF, D_OUT), lambda i: (0, 0)),
            pl.BlockSpec((_HALF, D_OUT), lambda i: (0, 0)),
            pl.BlockSpec((1, D_OUT), lambda i: (0, 0)),
        ],
        out_specs=pl.BlockSpec((_B3, D_OUT), lambda i: (i, 0)),
        out_shape=jax.ShapeDtypeStruct((N_EDGES, D_OUT), jnp.float32),
    )(g32, edge_attr, w1c_e, w1c_o, b1_e, b1_o, w2_e, w2_o, b2)


def kernel(node_attr, edge_attr, edge_index, edge_contact_attr,
           edge_contact_index, W1, b1, W2, b2):
    idx = edge_index.astype(jnp.int32)
    idx = jnp.pad(idx, ((0, 0), (0, _NPAD - N_EDGES)))
    sidx = idx[0].reshape(_GRID, _W)
    ridx = idx[1].reshape(_GRID, _W)

    ef = jnp.asarray(_EF)
    w1ab_p = W1[: 2 * D_FEAT, :][:, ef]
    p, q = _node_proj(node_attr, w1ab_p)
    g32 = _sc_gather_add(p, q, sidx, ridx)

    w1c = W1[2 * D_FEAT:, :]
    out = _epilogue(
        g32, edge_attr,
        w1c[:, 0::2], w1c[:, 1::2],
        b1[0::2].reshape(1, _HALF), b1[1::2].reshape(1, _HALF),
        W2[0::2, :], W2[1::2, :],
        b2.reshape(1, D_OUT))
    return (node_attr, out, edge_index, edge_contact_attr, edge_contact_index)
